# Initial kernel scaffold; baseline (speedup 1.0000x reference)
import jax
import jax.numpy as jnp
from jax.experimental import pallas as pl


def kernel(x, edge_index, edge_attr, We, be, W1, b1, W2, b2):
    raise NotImplementedError("write your pallas kernel here")


# speedup vs baseline: 1.9118x; 1.9118x over previous
import functools

import jax
import jax.numpy as jnp
from jax import lax
from jax.experimental import pallas as pl
from jax.experimental.pallas import tpu as pltpu
from jax.experimental.pallas import tpu_sc as plsc

N = 10000
E = 160000
D = 256
DH = 128
DE = 16

NSUB = 16
K = 80
EPT = E // NSUB
NCH = EPT // K
RPT_A = 632
RPT_LAST = N - 15 * RPT_A

EB = 2000
MB = 1000


def _edge_linear(edge_attr, We, be2):
    def body(a_ref, w_ref, b_ref, o_ref):
        o_ref[...] = jnp.dot(a_ref[...], w_ref[...],
                             preferred_element_type=jnp.float32) + b_ref[0]

    return pl.pallas_call(
        body,
        grid=(2, E // EB),
        in_specs=[
            pl.BlockSpec((EB, DE), lambda c, i: (i, 0)),
            pl.BlockSpec((DE, DH), lambda c, i: (0, c)),
            pl.BlockSpec((1, 1, DH), lambda c, i: (c, 0, 0)),
        ],
        out_specs=pl.BlockSpec((EB, DH), lambda c, i: (c * (E // EB) + i, 0)),
        out_shape=jax.ShapeDtypeStruct((2 * E, DH), jnp.float32),
    )(edge_attr, We, be2.reshape(2, 1, DH))


def _sc_aggregate(x2, eh, srcx, dst):
    mesh = plsc.VectorSubcoreMesh(core_axis_name="c", subcore_axis_name="s")

    @functools.partial(
        pl.kernel,
        out_type=jax.ShapeDtypeStruct((2 * N, DH), jnp.float32),
        mesh=mesh,
        scratch_types=[
            pltpu.VMEM_SHARED((N, DH), jnp.float32),
            pltpu.VMEM((K,), jnp.int32),
            pltpu.VMEM((K,), jnp.int32),
            pltpu.VMEM((K, DH), jnp.float32),
            pltpu.VMEM((K, DH), jnp.float32),
            pltpu.SemaphoreType.DMA,
        ],
    )
    def k(x2_hbm, eh_hbm, srcx_hbm, dst_hbm, out_hbm,
          acc, idx_s, idx_d, xbuf, ebuf, sem):
        c = lax.axis_index("c")
        s = lax.axis_index("s")

        def zbody(r, carry):
            for g in range(DH // 16):
                xbuf[r, pl.ds(g * 16, 16)] = jnp.zeros((16,), jnp.float32)
            return carry
        lax.fori_loop(0, K, zbody, 0)
        row0 = s * RPT_A

        def _zero_rows(nrows):
            full = nrows // K
            for kk in range(full):
                pltpu.sync_copy(xbuf, acc.at[pl.ds(row0 + kk * K, K)])
            r = nrows - full * K
            if r:
                pltpu.sync_copy(xbuf.at[pl.ds(0, r)],
                                acc.at[pl.ds(row0 + full * K, r)])

        pl.when(s < NSUB - 1)(lambda: _zero_rows(RPT_A))
        pl.when(s == NSUB - 1)(lambda: _zero_rows(RPT_LAST))
        plsc.subcore_barrier()

        ebase0 = s * EPT

        def chunk(i, carry):
            eb = ebase0 + i * K
            pltpu.sync_copy(srcx_hbm.at[pl.ds(c * E + eb, K)], idx_s)
            pltpu.sync_copy(dst_hbm.at[pl.ds(eb, K)], idx_d)
            cp = pltpu.async_copy(x2_hbm.at[idx_s], xbuf, sem)
            pltpu.sync_copy(eh_hbm.at[pl.ds(c * E + eb, K)], ebuf)
            cp.wait()

            def rbody(r, cr):
                for g in range(DH // 16):
                    sl = pl.ds(g * 16, 16)
                    ebuf[r, sl] = jnp.maximum(xbuf[r, sl] + ebuf[r, sl], 0.0)
                return cr
            lax.fori_loop(0, K, rbody, 0)

            pltpu.sync_copy(ebuf, acc.at[idx_d], add=True)
            return carry
        lax.fori_loop(0, NCH, chunk, 0)
        plsc.subcore_barrier()

        def _writeout(nrows):
            pltpu.sync_copy(acc.at[pl.ds(row0, nrows)],
                            out_hbm.at[pl.ds(c * N + row0, nrows)])

        pl.when(s < NSUB - 1)(lambda: _writeout(RPT_A))
        pl.when(s == NSUB - 1)(lambda: _writeout(RPT_LAST))

    return k(x2, eh, srcx, dst)


def _mlp(x, h2, W1, b1r, W2, b2r):
    def body(x_ref, al_ref, ar_ref, w1_ref, b1_ref, w2_ref, b2_ref, o_ref):
        h = x_ref[...] + jnp.concatenate([al_ref[...], ar_ref[...]], axis=1)
        t = jnp.maximum(
            jnp.dot(h, w1_ref[...], preferred_element_type=jnp.float32)
            + b1_ref[...], 0.0)
        o_ref[...] = jnp.dot(t, w2_ref[...],
                             preferred_element_type=jnp.float32) + b2_ref[...]

    return pl.pallas_call(
        body,
        grid=(N // MB,),
        in_specs=[
            pl.BlockSpec((MB, D), lambda i: (i, 0)),
            pl.BlockSpec((MB, DH), lambda i: (i, 0)),
            pl.BlockSpec((MB, DH), lambda i: (N // MB + i, 0)),
            pl.BlockSpec((D, D), lambda i: (0, 0)),
            pl.BlockSpec((1, D), lambda i: (0, 0)),
            pl.BlockSpec((D, D), lambda i: (0, 0)),
            pl.BlockSpec((1, D), lambda i: (0, 0)),
        ],
        out_specs=pl.BlockSpec((MB, D), lambda i: (i, 0)),
        out_shape=jax.ShapeDtypeStruct((N, D), jnp.float32),
    )(x, h2, h2, W1, b1r, W2, b2r)


def kernel(x, edge_index, edge_attr, We, be, W1, b1, W2, b2):
    src = edge_index[0].astype(jnp.int32)
    dst = edge_index[1].astype(jnp.int32)
    src2 = src * 2
    srcx = jnp.concatenate([src2, src2 + 1])
    x2 = x.reshape(2 * N, DH)

    eh = _edge_linear(edge_attr, We, be.reshape(2, DH))
    h2 = _sc_aggregate(x2, eh, srcx, dst)
    return _mlp(x, h2, W1, b1.reshape(1, D), W2, b2.reshape(1, D))


# speedup vs baseline: 2.9548x; 1.5455x over previous
import functools

import jax
import jax.numpy as jnp
from jax import lax
from jax.experimental import pallas as pl
from jax.experimental.pallas import tpu as pltpu
from jax.experimental.pallas import tpu_sc as plsc

N = 10000
E = 160000
D = 256
DH = 128
DE = 16

NSUB = 16
K = 80
EPT = E // NSUB
NCH = EPT // K
RPT_A = 632
RPT_LAST = N - 15 * RPT_A

EB = 2000
MB = 1000


def _edge_linear(edge_attr, We, be2):
    def body(a_ref, w_ref, b_ref, o_ref):
        o_ref[...] = jnp.dot(a_ref[...], w_ref[...],
                             preferred_element_type=jnp.float32) + b_ref[0]

    return pl.pallas_call(
        body,
        grid=(2, E // EB),
        in_specs=[
            pl.BlockSpec((EB, DE), lambda c, i: (i, 0)),
            pl.BlockSpec((DE, DH), lambda c, i: (0, c)),
            pl.BlockSpec((1, 1, DH), lambda c, i: (c, 0, 0)),
        ],
        out_specs=pl.BlockSpec((EB, DH), lambda c, i: (c * (E // EB) + i, 0)),
        out_shape=jax.ShapeDtypeStruct((2 * E, DH), jnp.float32),
    )(edge_attr, We, be2.reshape(2, 1, DH))


def _sc_aggregate(x2, eh, srcx4, dst3):
    mesh = plsc.VectorSubcoreMesh(core_axis_name="c", subcore_axis_name="s")

    @functools.partial(
        pl.kernel,
        out_type=jax.ShapeDtypeStruct((2 * N, DH), jnp.float32),
        mesh=mesh,
        scratch_types=[
            pltpu.VMEM_SHARED((N, DH), jnp.float32),
            pltpu.VMEM((4, K), jnp.int32),
            pltpu.VMEM((4, K), jnp.int32),
            pltpu.VMEM((2, K, DH), jnp.float32),
            pltpu.VMEM((2, K, DH), jnp.float32),
            pltpu.SemaphoreType.DMA,
            pltpu.SemaphoreType.DMA,
            pltpu.SemaphoreType.DMA,
            pltpu.SemaphoreType.DMA,
            pltpu.SemaphoreType.DMA,
            pltpu.SemaphoreType.DMA,
            pltpu.SemaphoreType.DMA,
            pltpu.SemaphoreType.DMA,
            pltpu.SemaphoreType.DMA,
        ],
    )
    def k(x2_hbm, eh_hbm, srcx_hbm, dst_hbm, out_hbm,
          acc, srcv, dstv, xbuf, ebuf, semx0, semx1, seme0, seme1, semsc,
          semi0, semi1, semi2, semi3):
        semx = [semx0, semx1]
        seme = [seme0, seme1]
        c = lax.axis_index("c")
        s = lax.axis_index("s")

        def zbody(r, carry):
            for g in range(DH // 16):
                xbuf[0, r, pl.ds(g * 16, 16)] = jnp.zeros((16,), jnp.float32)
            return carry
        lax.fori_loop(0, K, zbody, 0)
        row0 = s * RPT_A

        def _zero_rows(nrows):
            full = nrows // K
            for kk in range(full):
                pltpu.sync_copy(xbuf.at[0], acc.at[pl.ds(row0 + kk * K, K)])
            r = nrows - full * K
            if r:
                pltpu.sync_copy(xbuf.at[0, pl.ds(0, r)],
                                acc.at[pl.ds(row0 + full * K, r)])

        pl.when(s < NSUB - 1)(lambda: _zero_rows(RPT_A))
        pl.when(s == NSUB - 1)(lambda: _zero_rows(RPT_LAST))
        plsc.subcore_barrier()

        ebase0 = s * EPT
        semi = [semi0, semi1, semi2, semi3]
        row_hbm = c * NSUB + s

        def _idx_start(i, q):
            pltpu.async_copy(srcx_hbm.at[row_hbm, i], srcv.at[q], semi[q])
            pltpu.async_copy(dst_hbm.at[s, i], dstv.at[q], semi[q])

        def _idx_wait(q):
            pltpu.make_async_copy(srcx_hbm.at[0, 0], srcv.at[q], semi[q]).wait()
            pltpu.make_async_copy(dst_hbm.at[0, 0], dstv.at[q], semi[q]).wait()

        def _start(i, q, p):
            pltpu.async_copy(x2_hbm.at[srcv.at[q]], xbuf.at[p], semx[p])
            pltpu.async_copy(eh_hbm.at[pl.ds(c * E + ebase0 + i * K, K)],
                             ebuf.at[p], seme[p])

        _idx_start(0, 0)
        _idx_start(1, 1)
        _idx_wait(0)
        _start(0, 0, 0)

        def step(i, b, first=False):
            p = b % 2
            pn = 1 - p
            qn1 = (b + 1) % 4
            qn2 = (b + 2) % 4

            wait_sc = lambda: pltpu.make_async_copy(
                ebuf.at[pn], acc.at[dstv.at[pn]], semsc).wait()
            if first:
                pl.when(i > 0)(wait_sc)
            else:
                wait_sc()
            pl.when(i + 2 < NCH)(lambda: _idx_start(i + 2, qn2))

            def _pref():
                _idx_wait(qn1)
                _start(i + 1, qn1, pn)
            pl.when(i + 1 < NCH)(_pref)

            pltpu.make_async_copy(
                x2_hbm.at[srcv.at[b]], xbuf.at[p], semx[p]).wait()
            pltpu.make_async_copy(
                eh_hbm.at[pl.ds(0, K)], ebuf.at[p], seme[p]).wait()

            def rbody(r, cr):
                for g in range(DH // 16):
                    sl = pl.ds(g * 16, 16)
                    ebuf[p, r, sl] = jnp.maximum(
                        xbuf[p, r, sl] + ebuf[p, r, sl], 0.0)
                return cr
            lax.fori_loop(0, K, rbody, 0)

            pltpu.async_copy(ebuf.at[p], acc.at[dstv.at[b]], semsc, add=True)

        def quad(j, carry):
            i0 = j * 4
            step(i0, 0, first=True)
            step(i0 + 1, 1)
            step(i0 + 2, 2)
            step(i0 + 3, 3)
            return carry
        lax.fori_loop(0, NCH // 4, quad, 0)
        step(NCH - 1, (NCH - 1) % 4)

        pltpu.make_async_copy(
            ebuf.at[(NCH - 1) % 2], acc.at[dstv.at[0]], semsc).wait()
        plsc.subcore_barrier()

        def _writeout(nrows):
            pltpu.sync_copy(acc.at[pl.ds(row0, nrows)],
                            out_hbm.at[pl.ds(c * N + row0, nrows)])

        pl.when(s < NSUB - 1)(lambda: _writeout(RPT_A))
        pl.when(s == NSUB - 1)(lambda: _writeout(RPT_LAST))

    return k(x2, eh, srcx4, dst3)


def _mlp(x, h2, W1, b1r, W2, b2r):
    def body(x_ref, al_ref, ar_ref, w1_ref, b1_ref, w2_ref, b2_ref, o_ref):
        h = x_ref[...] + jnp.concatenate([al_ref[...], ar_ref[...]], axis=1)
        t = jnp.maximum(
            jnp.dot(h, w1_ref[...], preferred_element_type=jnp.float32)
            + b1_ref[...], 0.0)
        o_ref[...] = jnp.dot(t, w2_ref[...],
                             preferred_element_type=jnp.float32) + b2_ref[...]

    return pl.pallas_call(
        body,
        grid=(N // MB,),
        in_specs=[
            pl.BlockSpec((MB, D), lambda i: (i, 0)),
            pl.BlockSpec((MB, DH), lambda i: (i, 0)),
            pl.BlockSpec((MB, DH), lambda i: (N // MB + i, 0)),
            pl.BlockSpec((D, D), lambda i: (0, 0)),
            pl.BlockSpec((1, D), lambda i: (0, 0)),
            pl.BlockSpec((D, D), lambda i: (0, 0)),
            pl.BlockSpec((1, D), lambda i: (0, 0)),
        ],
        out_specs=pl.BlockSpec((MB, D), lambda i: (i, 0)),
        out_shape=jax.ShapeDtypeStruct((N, D), jnp.float32),
    )(x, h2, h2, W1, b1r, W2, b2r)


def kernel(x, edge_index, edge_attr, We, be, W1, b1, W2, b2):
    src = edge_index[0].astype(jnp.int32)
    dst = edge_index[1].astype(jnp.int32)
    src2 = src * 2
    srcx4 = jnp.concatenate([src2, src2 + 1]).reshape(2 * NSUB, NCH, K)
    dst3 = dst.reshape(NSUB, NCH, K)
    x2 = x.reshape(2 * N, DH)

    eh = _edge_linear(edge_attr, We, be.reshape(2, DH))
    h2 = _sc_aggregate(x2, eh, srcx4, dst3)
    return _mlp(x, h2, W1, b1.reshape(1, D), W2, b2.reshape(1, D))
